# baseline (speedup 1.0000x reference)
import jax
import jax.numpy as jnp
from jax.experimental import pallas as pl
from jax.experimental.pallas import tpu as pltpu

E = 16
TOP_K = 8
D = 2048
F = 1408
B = 32
C = 704
NF = F // C


def _ffn_kernel(rl_ref, xt_ref, w1_ref, w1u_ref, w2_ref, out_ref,
                xt_bf, combt, acct, mixed):
    e = pl.program_id(0)
    f = pl.program_id(1)

    @pl.when((e == 0) & (f == 0))
    def _init():
        xt_bf[:, :] = xt_ref[:, :]
        acct[:, :] = jnp.zeros((D, B), jnp.float32)
        logits = rl_ref[:, :]
        vals = logits
        sel = jnp.zeros((B, E), jnp.float32)
        idx = jax.lax.broadcasted_iota(jnp.int32, (B, E), 1)
        for _ in range(TOP_K):
            am = jnp.argmax(vals, axis=1)
            first = idx == am[:, None]
            sel = jnp.where(first, 1.0, sel)
            vals = jnp.where(first, -jnp.inf, vals)
        mx = jnp.max(logits, axis=1, keepdims=True)
        ew = jnp.exp(logits - mx) * sel
        w = ew / jnp.sum(ew, axis=1, keepdims=True)
        combt[:, :] = w.T

    w1c = w1_ref[0]
    w1uc = w1u_ref[0]
    xtb = xt_bf[:, :]
    gt = jax.lax.dot_general(w1c, xtb, (((1,), (0,)), ((), ())),
                             preferred_element_type=jnp.float32)
    ut = jax.lax.dot_general(w1uc, xtb, (((1,), (0,)), ((), ())),
                             preferred_element_type=jnp.float32)
    cw = combt[pl.ds(e, 1), :]
    mt = gt * jax.lax.logistic(gt) * ut * cw
    mixed[pl.ds(f * C, C), :] = mt

    @pl.when(f == NF - 1)
    def _down():
        acct[:, :] += jax.lax.dot_general(
            w2_ref[0], mixed[:, :], (((1,), (0,)), ((), ())),
            preferred_element_type=jnp.float32)

    @pl.when((e == E - 1) & (f == NF - 1))
    def _fin():
        out_ref[:, :] = acct[:, :]


def kernel(x, router_logits, w1, w1_up, w2):
    if x.ndim == 2:
        x = x[:, None, :]
    curr = x[:, -1, :]
    outt = pl.pallas_call(
        _ffn_kernel,
        grid=(E, NF),
        in_specs=[
            pl.BlockSpec((B, E), lambda e, f: (0, 0)),
            pl.BlockSpec((D, B), lambda e, f: (0, 0)),
            pl.BlockSpec((1, C, D), lambda e, f: (e, f, 0)),
            pl.BlockSpec((1, C, D), lambda e, f: (e, f, 0)),
            pl.BlockSpec((1, D, F), lambda e, f: (e, 0, 0)),
        ],
        out_specs=pl.BlockSpec((D, B), lambda e, f: (0, 0)),
        out_shape=jax.ShapeDtypeStruct((D, B), jnp.float32),
        scratch_shapes=[
            pltpu.VMEM((D, B), jnp.float32),
            pltpu.VMEM((E, B), jnp.float32),
            pltpu.VMEM((D, B), jnp.float32),
            pltpu.VMEM((F, B), jnp.float32),
        ],
        compiler_params=pltpu.CompilerParams(
            dimension_semantics=("arbitrary", "arbitrary")),
    )(router_logits, curr.T, w1, w1_up, w2)
    return outt.T.reshape(x.shape[0], 1, D)


# speedup vs baseline: 1.0168x; 1.0168x over previous
import jax
import jax.numpy as jnp
from jax.experimental import pallas as pl
from jax.experimental.pallas import tpu as pltpu

E = 16
TOP_K = 8
D = 2048
F = 1408
B = 32
C = 512
NF = -(-F // C)
R = F - (NF - 1) * C


def _ffn_kernel(rl_ref, xt_ref, w1_ref, w1u_ref, w2_ref, out_ref,
                xt_v, combt, acct):
    e = pl.program_id(0)
    f = pl.program_id(1)

    @pl.when((e == 0) & (f == 0))
    def _init():
        xt_v[:, :] = xt_ref[:, :]
        acct[:, :] = jnp.zeros((D, B), jnp.float32)
        logits = rl_ref[:, :]
        vals = logits
        sel = jnp.zeros((B, E), jnp.float32)
        idx = jax.lax.broadcasted_iota(jnp.int32, (B, E), 1)
        for _ in range(TOP_K):
            am = jnp.argmax(vals, axis=1)
            first = idx == am[:, None]
            sel = jnp.where(first, 1.0, sel)
            vals = jnp.where(first, -jnp.inf, vals)
        mx = jnp.max(logits, axis=1, keepdims=True)
        ew = jnp.exp(logits - mx) * sel
        w = ew / jnp.sum(ew, axis=1, keepdims=True)
        combt[:, :] = w.T

    xtb = xt_v[:, :]
    gt = jax.lax.dot_general(w1_ref[0], xtb, (((1,), (0,)), ((), ())),
                             preferred_element_type=jnp.float32)
    ut = jax.lax.dot_general(w1u_ref[0], xtb, (((1,), (0,)), ((), ())),
                             preferred_element_type=jnp.float32)
    cw = combt[pl.ds(e, 1), :]
    mt = gt * jax.lax.logistic(gt) * ut * cw

    @pl.when(f < NF - 1)
    def _down_full():
        acct[:, :] += jax.lax.dot_general(
            w2_ref[0], mt, (((1,), (0,)), ((), ())),
            preferred_element_type=jnp.float32)

    @pl.when(f == NF - 1)
    def _down_tail():
        acct[:, :] += jax.lax.dot_general(
            w2_ref[0][:, :R], mt[:R], (((1,), (0,)), ((), ())),
            preferred_element_type=jnp.float32)

    @pl.when((e == E - 1) & (f == NF - 1))
    def _fin():
        out_ref[:, :] = acct[:, :]


def kernel(x, router_logits, w1, w1_up, w2):
    if x.ndim == 2:
        x = x[:, None, :]
    curr = x[:, -1, :]
    outt = pl.pallas_call(
        _ffn_kernel,
        grid=(E, NF),
        in_specs=[
            pl.BlockSpec((B, E), lambda e, f: (0, 0)),
            pl.BlockSpec((D, B), lambda e, f: (0, 0)),
            pl.BlockSpec((1, C, D), lambda e, f: (e, f, 0)),
            pl.BlockSpec((1, C, D), lambda e, f: (e, f, 0)),
            pl.BlockSpec((1, D, C), lambda e, f: (e, 0, f)),
        ],
        out_specs=pl.BlockSpec((D, B), lambda e, f: (0, 0)),
        out_shape=jax.ShapeDtypeStruct((D, B), jnp.float32),
        scratch_shapes=[
            pltpu.VMEM((D, B), jnp.float32),
            pltpu.VMEM((E, B), jnp.float32),
            pltpu.VMEM((D, B), jnp.float32),
        ],
        compiler_params=pltpu.CompilerParams(
            dimension_semantics=("arbitrary", "arbitrary")),
    )(router_logits, curr.T, w1, w1_up, w2)
    return outt.T.reshape(x.shape[0], 1, D)


# speedup vs baseline: 1.0265x; 1.0096x over previous
import jax
import jax.numpy as jnp
from jax.experimental import pallas as pl
from jax.experimental.pallas import tpu as pltpu

E = 16
TOP_K = 8
D = 2048
F = 1408
B = 32
NF = 2
C = F // NF
DC = D // NF


def _ffn_kernel(rl_ref, xt_ref, w1_ref, w1u_ref, w2_ref, out_ref,
                xt_v, combt, acct, mixa, mixb):
    e = pl.program_id(0)
    f = pl.program_id(1)

    @pl.when((e == 0) & (f == 0))
    def _init():
        xt_v[:, :] = xt_ref[:, :]
        acct[:, :] = jnp.zeros((D, B), jnp.float32)
        logits = rl_ref[:, :]
        vals = logits
        sel = jnp.zeros((B, E), jnp.float32)
        idx = jax.lax.broadcasted_iota(jnp.int32, (B, E), 1)
        for _ in range(TOP_K):
            am = jnp.argmax(vals, axis=1)
            first = idx == am[:, None]
            sel = jnp.where(first, 1.0, sel)
            vals = jnp.where(first, -jnp.inf, vals)
        mx = jnp.max(logits, axis=1, keepdims=True)
        ew = jnp.exp(logits - mx) * sel
        w = ew / jnp.sum(ew, axis=1, keepdims=True)
        combt[:, :] = w.T

    @pl.when(e < E)
    def _gate_up():
        xtb = xt_v[:, :]
        gt = jax.lax.dot_general(w1_ref[0], xtb, (((1,), (0,)), ((), ())),
                                 preferred_element_type=jnp.float32)
        ut = jax.lax.dot_general(w1u_ref[0], xtb, (((1,), (0,)), ((), ())),
                                 preferred_element_type=jnp.float32)
        cw = combt[pl.ds(e, 1), :]
        mt = gt * jax.lax.logistic(gt) * ut * cw

        @pl.when(e % 2 == 0)
        def _():
            mixa[pl.ds(f * C, C), :] = mt

        @pl.when(e % 2 == 1)
        def _():
            mixb[pl.ds(f * C, C), :] = mt

    @pl.when(e > 0)
    def _down():
        @pl.when(e % 2 == 1)
        def _():
            acct[pl.ds(f * DC, DC), :] += jax.lax.dot_general(
                w2_ref[0], mixa[:, :], (((1,), (0,)), ((), ())),
                preferred_element_type=jnp.float32)

        @pl.when(e % 2 == 0)
        def _():
            acct[pl.ds(f * DC, DC), :] += jax.lax.dot_general(
                w2_ref[0], mixb[:, :], (((1,), (0,)), ((), ())),
                preferred_element_type=jnp.float32)

    @pl.when((e == E) & (f == NF - 1))
    def _fin():
        out_ref[:, :] = acct[:, :]


def kernel(x, router_logits, w1, w1_up, w2):
    if x.ndim == 2:
        x = x[:, None, :]
    curr = x[:, -1, :]
    outt = pl.pallas_call(
        _ffn_kernel,
        grid=(E + 1, NF),
        in_specs=[
            pl.BlockSpec((B, E), lambda e, f: (0, 0)),
            pl.BlockSpec((D, B), lambda e, f: (0, 0)),
            pl.BlockSpec((1, C, D),
                         lambda e, f: (jnp.minimum(e, E - 1),
                                       jnp.where(e < E, f, NF - 1), 0)),
            pl.BlockSpec((1, C, D),
                         lambda e, f: (jnp.minimum(e, E - 1),
                                       jnp.where(e < E, f, NF - 1), 0)),
            pl.BlockSpec((1, DC, F),
                         lambda e, f: (jnp.maximum(e - 1, 0),
                                       jnp.where(e == 0, 0, f), 0)),
        ],
        out_specs=pl.BlockSpec((D, B), lambda e, f: (0, 0)),
        out_shape=jax.ShapeDtypeStruct((D, B), jnp.float32),
        scratch_shapes=[
            pltpu.VMEM((D, B), jnp.float32),
            pltpu.VMEM((E, B), jnp.float32),
            pltpu.VMEM((D, B), jnp.float32),
            pltpu.VMEM((F, B), jnp.float32),
            pltpu.VMEM((F, B), jnp.float32),
        ],
        compiler_params=pltpu.CompilerParams(
            dimension_semantics=("arbitrary", "arbitrary")),
    )(router_logits, curr.T, w1, w1_up, w2)
    return outt.T.reshape(x.shape[0], 1, D)
